# baseline (speedup 1.0000x reference)
import functools

import jax
import jax.numpy as jnp
from jax.experimental import pallas as pl
from jax.experimental.pallas import tpu as pltpu

N_EMBD = 768
D_FF = 4 * N_EMBD
NUM_EXPERTS = 8
TOP_K = 2
BLK = 256
NPAIR = 2048 * TOP_K
NBLK = NPAIR // BLK + NUM_EXPERTS
P = NBLK * BLK


def _mlp_block(boff_ref, nb_ref, xs_ref, w1_ref, b1_ref, w2_ref, b2_ref, y_ref):
    e = pl.program_id(0)
    b = pl.program_id(1)

    @pl.when(b < nb_ref[e])
    def _():
        xb = xs_ref[...].astype(jnp.bfloat16)
        w1 = w1_ref[0].astype(jnp.bfloat16)
        h = jnp.dot(xb, w1, preferred_element_type=jnp.float32) + b1_ref[0]
        h = jnp.maximum(h, 0.0).astype(jnp.bfloat16)
        w2 = w2_ref[0].astype(jnp.bfloat16)
        y = jnp.dot(h, w2, preferred_element_type=jnp.float32) + b2_ref[0]
        y_ref[...] = y


@functools.partial(jax.jit, static_argnames=())
def _grouped_mlp(boff, nb, xs, W1, b1, W2, b2):
    grid = (NUM_EXPERTS, NUM_EXPERTS)

    def xs_map(e, b, boff_ref, nb_ref):
        return (jnp.where(b < nb_ref[e], boff_ref[e] + b, 0), 0)

    def y_map(e, b, boff_ref, nb_ref):
        return (jnp.where(b < nb_ref[e], boff_ref[e] + b, NBLK), 0)

    def w_map(e, b, boff_ref, nb_ref):
        return (e, 0, 0)

    def bias_map(e, b, boff_ref, nb_ref):
        return (e, 0, 0)

    return pl.pallas_call(
        _mlp_block,
        grid_spec=pltpu.PrefetchScalarGridSpec(
            num_scalar_prefetch=2,
            grid=grid,
            in_specs=[
                pl.BlockSpec((BLK, N_EMBD), xs_map),
                pl.BlockSpec((1, N_EMBD, D_FF), w_map),
                pl.BlockSpec((1, 1, D_FF), bias_map),
                pl.BlockSpec((1, D_FF, N_EMBD), w_map),
                pl.BlockSpec((1, 1, N_EMBD), bias_map),
            ],
            out_specs=pl.BlockSpec((BLK, N_EMBD), y_map),
        ),
        out_shape=jax.ShapeDtypeStruct((P + BLK, N_EMBD), jnp.float32),
        compiler_params=pltpu.CompilerParams(
            dimension_semantics=("arbitrary", "arbitrary"),
        ),
    )(boff, nb, xs, W1, b1.reshape(NUM_EXPERTS, 1, D_FF), W2,
      b2.reshape(NUM_EXPERTS, 1, N_EMBD))


def kernel(x, Wg, bg, W1, b1, W2, b2):
    Bb, Tt, C = x.shape
    x_flat = x.reshape(-1, C)

    logits = x_flat @ Wg + bg
    gates2, idx2 = jax.lax.top_k(logits, TOP_K)
    g2 = jax.nn.softmax(gates2, axis=-1)
    e_flat = idx2.reshape(-1)

    onehot = (e_flat[:, None] == jnp.arange(NUM_EXPERTS)[None, :]).astype(jnp.int32)
    csum = jnp.cumsum(onehot, axis=0)
    counts = csum[-1]
    rank = jnp.take_along_axis(csum, e_flat[:, None], axis=1)[:, 0] - 1
    nb = (counts + BLK - 1) // BLK
    boff = jnp.concatenate([jnp.zeros((1,), jnp.int32),
                            jnp.cumsum(nb)[:-1].astype(jnp.int32)])
    nb = nb.astype(jnp.int32)
    pos = boff[e_flat] * BLK + rank

    perm = jnp.argsort(e_flat, stable=True)
    coff = jnp.concatenate([jnp.zeros((1,), jnp.int32),
                            jnp.cumsum(counts)[:-1].astype(jnp.int32)])
    slots = jnp.arange(P, dtype=jnp.int32)
    es = jnp.sum(slots[:, None] >= (boff * BLK)[None, :], axis=1).astype(jnp.int32) - 1
    es = jnp.clip(es, 0, NUM_EXPERTS - 1)
    r = slots - boff[es] * BLK
    valid = r < counts[es]
    src = coff[es] + jnp.minimum(r, jnp.maximum(counts[es] - 1, 0))
    pair_at = perm[src]
    sorted_tok = jnp.where(valid, pair_at // TOP_K, 0)

    xs = x_flat[sorted_tok]

    y = _grouped_mlp(boff, nb, xs, W1, b1, W2, b2)

    p_mat = pos.reshape(Tt, TOP_K)
    out = (g2[:, 0:1] * y[p_mat[:, 0]] + g2[:, 1:2] * y[p_mat[:, 1]])
    return out.reshape(Bb, Tt, C)


# speedup vs baseline: 1.0510x; 1.0510x over previous
import functools

import jax
import jax.numpy as jnp
from jax.experimental import pallas as pl
from jax.experimental.pallas import tpu as pltpu

N_EMBD = 768
D_FF = 4 * N_EMBD
NUM_EXPERTS = 8
TOP_K = 2
BLK = 256
NPAIR = 2048 * TOP_K
NBLK = NPAIR // BLK + NUM_EXPERTS
P = NBLK * BLK


def _mlp_block(be_ref, nact_ref, xs_ref, w1_ref, b1_ref, w2_ref, b2_ref,
               y_ref, w1s_ref, w2s_ref):
    g = pl.program_id(0)

    @pl.when(g < nact_ref[0])
    def _():
        new_expert = jnp.logical_or(
            g == 0, be_ref[g] != be_ref[jnp.maximum(g - 1, 0)])

        @pl.when(new_expert)
        def _():
            w1s_ref[...] = w1_ref[0].astype(jnp.bfloat16)
            w2s_ref[...] = w2_ref[0].astype(jnp.bfloat16)

        xb = xs_ref[...].astype(jnp.bfloat16)
        h = jnp.dot(xb, w1s_ref[...], preferred_element_type=jnp.float32)
        h = jnp.maximum(h + b1_ref[0], 0.0).astype(jnp.bfloat16)
        y = jnp.dot(h, w2s_ref[...], preferred_element_type=jnp.float32)
        y_ref[...] = y + b2_ref[0]


@functools.partial(jax.jit, static_argnames=())
def _grouped_mlp(be, nact, xs, W1, b1, W2, b2):
    def w_map(g, be_ref, nact_ref):
        return (be_ref[g], 0, 0)

    return pl.pallas_call(
        _mlp_block,
        grid_spec=pltpu.PrefetchScalarGridSpec(
            num_scalar_prefetch=2,
            grid=(NBLK,),
            in_specs=[
                pl.BlockSpec((BLK, N_EMBD), lambda g, be, na: (g, 0)),
                pl.BlockSpec((1, N_EMBD, D_FF), w_map),
                pl.BlockSpec((1, 1, D_FF), w_map),
                pl.BlockSpec((1, D_FF, N_EMBD), w_map),
                pl.BlockSpec((1, 1, N_EMBD), w_map),
            ],
            out_specs=pl.BlockSpec((BLK, N_EMBD), lambda g, be, na: (g, 0)),
            scratch_shapes=[
                pltpu.VMEM((N_EMBD, D_FF), jnp.bfloat16),
                pltpu.VMEM((D_FF, N_EMBD), jnp.bfloat16),
            ],
        ),
        out_shape=jax.ShapeDtypeStruct((P, N_EMBD), jnp.float32),
        compiler_params=pltpu.CompilerParams(
            dimension_semantics=("arbitrary",),
        ),
    )(be, nact, xs, W1, b1.reshape(NUM_EXPERTS, 1, D_FF), W2,
      b2.reshape(NUM_EXPERTS, 1, N_EMBD))


def kernel(x, Wg, bg, W1, b1, W2, b2):
    Bb, Tt, C = x.shape
    x_flat = x.reshape(-1, C)

    logits = x_flat @ Wg + bg
    gates2, idx2 = jax.lax.top_k(logits, TOP_K)
    g2 = jax.nn.softmax(gates2, axis=-1)
    e_flat = idx2.reshape(-1)

    onehot = (e_flat[:, None] == jnp.arange(NUM_EXPERTS)[None, :]).astype(jnp.int32)
    csum = jnp.cumsum(onehot, axis=0)
    counts = csum[-1]
    rank = jnp.take_along_axis(csum, e_flat[:, None], axis=1)[:, 0] - 1
    nb = (counts + BLK - 1) // BLK
    boff = jnp.concatenate([jnp.zeros((1,), jnp.int32),
                            jnp.cumsum(nb)[:-1].astype(jnp.int32)])
    nb = nb.astype(jnp.int32)
    pos = boff[e_flat] * BLK + rank

    perm = jnp.argsort(e_flat, stable=True)
    coff = jnp.concatenate([jnp.zeros((1,), jnp.int32),
                            jnp.cumsum(counts)[:-1].astype(jnp.int32)])
    slots = jnp.arange(P, dtype=jnp.int32)
    es = jnp.sum(slots[:, None] >= (boff * BLK)[None, :], axis=1).astype(jnp.int32) - 1
    es = jnp.clip(es, 0, NUM_EXPERTS - 1)
    r = slots - boff[es] * BLK
    valid = r < counts[es]
    src = coff[es] + jnp.minimum(r, jnp.maximum(counts[es] - 1, 0))
    pair_at = perm[src]
    sorted_tok = jnp.where(valid, pair_at // TOP_K, 0)

    bb = jnp.cumsum(nb).astype(jnp.int32)
    nact = bb[-1:]
    garr = jnp.arange(NBLK, dtype=jnp.int32)
    be = jnp.sum(garr[:, None] >= bb[None, :], axis=1).astype(jnp.int32)
    be = jnp.clip(be, 0, NUM_EXPERTS - 1)
    be = jnp.where(garr < nact, be, be[jnp.maximum(nact - 1, 0)])

    xs = x_flat[sorted_tok]

    y = _grouped_mlp(be, nact, xs, W1, b1, W2, b2)

    p_mat = pos.reshape(Tt, TOP_K)
    out = (g2[:, 0:1] * y[p_mat[:, 0]] + g2[:, 1:2] * y[p_mat[:, 1]])
    return out.reshape(Bb, Tt, C)
